# baseline (speedup 1.0000x reference)
import jax
import jax.numpy as jnp
from jax import lax
from jax.experimental import pallas as pl
from jax.experimental.pallas import tpu as pltpu
from jax.experimental.pallas import tpu_sc as plsc

_BLOCK_ROWS = 16


def kernel(x, enc_weight):
    seq_len = x.shape[1]
    d = enc_weight.shape[1]
    mesh = plsc.VectorSubcoreMesh(core_axis_name="c", subcore_axis_name="s")

    def body(w_hbm, o_hbm):
        def copy_block(in_vmem, out_vmem):
            @pl.loop(0, _BLOCK_ROWS)
            def _(r):
                @pl.loop(0, d, step=16)
                def _(c):
                    slc = (pl.ds(r, 1), pl.ds(c, 16))
                    out_vmem.at[*slc][...] = in_vmem.at[*slc][...]

        pltpu.emit_pipeline(
            copy_block,
            grid=(seq_len // _BLOCK_ROWS,),
            in_specs=[pl.BlockSpec((_BLOCK_ROWS, d), lambda i: (i, 0))],
            out_specs=[pl.BlockSpec((_BLOCK_ROWS, d), lambda i: (i, 0))],
            core_axis_name=("c", "s"),
            dimension_semantics=(pltpu.PARALLEL,),
        )(w_hbm, o_hbm)

    return pl.kernel(
        body,
        out_type=jax.ShapeDtypeStruct((seq_len, d), enc_weight.dtype),
        mesh=mesh,
    )(enc_weight)


# speedup vs baseline: 1.8982x; 1.8982x over previous
import jax
import jax.numpy as jnp
from jax import lax
from jax.experimental import pallas as pl
from jax.experimental.pallas import tpu as pltpu
from jax.experimental.pallas import tpu_sc as plsc

_CHUNK = 32


def kernel(x, enc_weight):
    seq_len = x.shape[1]
    d = enc_weight.shape[1]
    dtype = enc_weight.dtype

    mesh = plsc.VectorSubcoreMesh(core_axis_name="c", subcore_axis_name="s")
    num_workers = mesh.num_cores * mesh.num_subcores
    rows_per_w = seq_len // num_workers
    assert rows_per_w * num_workers == seq_len
    n_chunks = rows_per_w // _CHUNK
    assert n_chunks * _CHUNK == rows_per_w and n_chunks >= 2

    def body(w_hbm, o_hbm, buf0, buf1, si0, si1, so0, so1):
        wid = lax.axis_index("s") * mesh.num_cores + lax.axis_index("c")
        base = wid * rows_per_w
        bufs = (buf0, buf1)
        in_sems = (si0, si1)
        out_sems = (so0, so1)

        def load(i):
            return pltpu.async_copy(
                w_hbm.at[pl.ds(base + i * _CHUNK, _CHUNK)],
                bufs[i % 2], in_sems[i % 2])

        def store(i):
            return pltpu.async_copy(
                bufs[i % 2],
                o_hbm.at[pl.ds(base + i * _CHUNK, _CHUNK)],
                out_sems[i % 2])

        in_h = [None, None]
        out_h = [None, None]
        in_h[0] = load(0)
        for i in range(n_chunks):
            b = i % 2
            if i + 1 < n_chunks:
                nb = (i + 1) % 2
                if out_h[nb] is not None:
                    out_h[nb].wait()
                in_h[nb] = load(i + 1)
            in_h[b].wait()
            out_h[b] = store(i)
        out_h[(n_chunks - 1) % 2].wait()
        if n_chunks >= 2:
            out_h[n_chunks % 2].wait()

    return pl.kernel(
        body,
        out_type=jax.ShapeDtypeStruct((seq_len, d), dtype),
        mesh=mesh,
        scratch_types=[
            pltpu.VMEM((_CHUNK, d), dtype),
            pltpu.VMEM((_CHUNK, d), dtype),
            pltpu.SemaphoreType.DMA,
            pltpu.SemaphoreType.DMA,
            pltpu.SemaphoreType.DMA,
            pltpu.SemaphoreType.DMA,
        ],
    )(enc_weight)


# speedup vs baseline: 1.9561x; 1.0305x over previous
import jax
import jax.numpy as jnp
from jax import lax
from jax.experimental import pallas as pl
from jax.experimental.pallas import tpu as pltpu
from jax.experimental.pallas import tpu_sc as plsc

_CHUNK = 32
_NBUF = 3


def kernel(x, enc_weight):
    seq_len = x.shape[1]
    d = enc_weight.shape[1]
    dtype = enc_weight.dtype

    mesh = plsc.VectorSubcoreMesh(core_axis_name="c", subcore_axis_name="s")
    num_workers = mesh.num_cores * mesh.num_subcores
    rows_per_w = seq_len // num_workers
    assert rows_per_w * num_workers == seq_len
    n_chunks = rows_per_w // _CHUNK
    assert n_chunks * _CHUNK == rows_per_w and n_chunks >= _NBUF

    def body(w_hbm, o_hbm, *scratch):
        bufs = scratch[:_NBUF]
        in_sems = scratch[_NBUF:2 * _NBUF]
        out_sems = scratch[2 * _NBUF:]
        wid = lax.axis_index("s") * mesh.num_cores + lax.axis_index("c")
        base = wid * rows_per_w

        def load(i):
            return pltpu.async_copy(
                w_hbm.at[pl.ds(base + i * _CHUNK, _CHUNK)],
                bufs[i % _NBUF], in_sems[i % _NBUF])

        def store(i):
            return pltpu.async_copy(
                bufs[i % _NBUF],
                o_hbm.at[pl.ds(base + i * _CHUNK, _CHUNK)],
                out_sems[i % _NBUF])

        in_h = {}
        out_h = {}
        waited = set()
        for i in range(_NBUF - 1):
            in_h[i] = load(i)
        for i in range(n_chunks):
            j = i + _NBUF - 1
            if j < n_chunks:
                prev = j - _NBUF
                if prev >= 0:
                    out_h[prev].wait()
                    waited.add(prev)
                in_h[j] = load(j)
            in_h[i].wait()
            out_h[i] = store(i)
        for i in range(n_chunks):
            if i not in waited:
                out_h[i].wait()

    return pl.kernel(
        body,
        out_type=jax.ShapeDtypeStruct((seq_len, d), dtype),
        mesh=mesh,
        scratch_types=(
            [pltpu.VMEM((_CHUNK, d), dtype) for _ in range(_NBUF)]
            + [pltpu.SemaphoreType.DMA for _ in range(2 * _NBUF)]
        ),
    )(enc_weight)


# speedup vs baseline: 2.0037x; 1.0243x over previous
import jax
import jax.numpy as jnp
from jax import lax
from jax.experimental import pallas as pl
from jax.experimental.pallas import tpu as pltpu
from jax.experimental.pallas import tpu_sc as plsc

_CHUNK = 16
_NBUF = 6


def kernel(x, enc_weight):
    seq_len = x.shape[1]
    d = enc_weight.shape[1]
    dtype = enc_weight.dtype

    mesh = plsc.VectorSubcoreMesh(core_axis_name="c", subcore_axis_name="s")
    num_workers = mesh.num_cores * mesh.num_subcores
    rows_per_w = seq_len // num_workers
    assert rows_per_w * num_workers == seq_len
    n_chunks = rows_per_w // _CHUNK
    assert n_chunks * _CHUNK == rows_per_w and n_chunks >= _NBUF

    def body(w_hbm, o_hbm, *scratch):
        bufs = scratch[:_NBUF]
        in_sems = scratch[_NBUF:2 * _NBUF]
        out_sems = scratch[2 * _NBUF:]
        wid = lax.axis_index("s") * mesh.num_cores + lax.axis_index("c")
        base = wid * rows_per_w

        def load(i):
            return pltpu.async_copy(
                w_hbm.at[pl.ds(base + i * _CHUNK, _CHUNK)],
                bufs[i % _NBUF], in_sems[i % _NBUF])

        def store(i):
            return pltpu.async_copy(
                bufs[i % _NBUF],
                o_hbm.at[pl.ds(base + i * _CHUNK, _CHUNK)],
                out_sems[i % _NBUF])

        in_h = {}
        out_h = {}
        waited = set()
        for i in range(_NBUF - 1):
            in_h[i] = load(i)
        for i in range(n_chunks):
            j = i + _NBUF - 1
            if j < n_chunks:
                prev = j - _NBUF
                if prev >= 0:
                    out_h[prev].wait()
                    waited.add(prev)
                in_h[j] = load(j)
            in_h[i].wait()
            out_h[i] = store(i)
        for i in range(n_chunks):
            if i not in waited:
                out_h[i].wait()

    return pl.kernel(
        body,
        out_type=jax.ShapeDtypeStruct((seq_len, d), dtype),
        mesh=mesh,
        scratch_types=(
            [pltpu.VMEM((_CHUNK, d), dtype) for _ in range(_NBUF)]
            + [pltpu.SemaphoreType.DMA for _ in range(2 * _NBUF)]
        ),
    )(enc_weight)
